# Initial kernel scaffold; baseline (speedup 1.0000x reference)
import jax
import jax.numpy as jnp
from jax import lax
from jax.experimental import pallas as pl
from jax.experimental.pallas import tpu as pltpu
from jax.experimental.pallas import tpu_sc as plsc

N_NODES = 10000
N_EDGES = 320000
D = 128
N_MOL = 64

BE = 2000
GRID = N_EDGES // BE
HN = 64
N_HPAD = GRID * HN

NC = 2
NS = 16
NW = NC * NS
EPW = N_EDGES // NW
CH = 125
NCH = EPW // CH
NPAD = 10240
RPW = NPAD // NS


def _tc_body(h_ref, m_ref, v_ref, we_ref, wf_ref, b_ref, bid_ref,
             e_ref, fv_ref):
    f = jnp.dot(m_ref[...], wf_ref[...],
                preferred_element_type=jnp.float32) + b_ref[0, 0]
    v4 = jnp.concatenate(
        [v_ref[...], jnp.zeros((BE, 1), jnp.float32)], axis=1)
    fv_ref[...] = f * v4

    e = jnp.dot(h_ref[...], we_ref[...],
                preferred_element_type=jnp.float32)
    ids = bid_ref[0, 0, :]
    seg = lax.broadcasted_iota(jnp.int32, (N_MOL, HN), 0)
    onehot = (seg == ids[None, :]).astype(jnp.float32)
    contrib = jnp.dot(onehot, e, preferred_element_type=jnp.float32)

    @pl.when(pl.program_id(0) == 0)
    def _():
        e_ref[...] = jnp.zeros_like(e_ref)

    e_ref[...] += contrib


def _sc_body(fv_hbm, idx_hbm, zeros_hbm, out_hbm, fv_v, idx_v, acc_sh):
    c = lax.axis_index("c")
    s = lax.axis_index("s")
    w = c * NS + s
    base = w * EPW

    pltpu.sync_copy(fv_hbm.at[pl.ds(base, EPW)], fv_v)
    pltpu.sync_copy(idx_hbm.at[w], idx_v)

    @pl.when(s == 0)
    def _():
        pltpu.sync_copy(zeros_hbm, acc_sh)

    plsc.subcore_barrier()

    def chunk(j, carry):
        pltpu.sync_copy(fv_v.at[pl.ds(j * CH, CH)],
                        acc_sh.at[idx_v.at[j]], add=True)
        return carry

    lax.fori_loop(0, NCH, chunk, 0)

    plsc.subcore_barrier()

    pltpu.sync_copy(acc_sh.at[pl.ds(s * RPW, RPW)],
                    out_hbm.at[c, pl.ds(s * RPW, RPW)])


def _comb_body(p_ref, o_ref):
    o_ref[...] = p_ref[0, :N_NODES, :3] + p_ref[1, :N_NODES, :3]


def kernel(h_energy, m_forces, V_st, W_energy, W_forces, b_forces,
           batch_ids, idx_t):
    h_pad = jnp.pad(h_energy, ((0, N_HPAD - N_NODES), (0, 0)))
    bid_pad = jnp.pad(batch_ids, (0, N_HPAD - N_NODES)).astype(jnp.int32)
    bid_pad = bid_pad.reshape(GRID, 1, HN)
    b2 = b_forces.reshape(1, 1)

    energy2, fv = pl.pallas_call(
        _tc_body,
        grid=(GRID,),
        in_specs=[
            pl.BlockSpec((HN, D), lambda i: (i, 0)),
            pl.BlockSpec((BE, D), lambda i: (i, 0)),
            pl.BlockSpec((BE, 3), lambda i: (i, 0)),
            pl.BlockSpec((D, 1), lambda i: (0, 0)),
            pl.BlockSpec((D, 1), lambda i: (0, 0)),
            pl.BlockSpec((1, 1), lambda i: (0, 0)),
            pl.BlockSpec((1, 1, HN), lambda i: (i, 0, 0)),
        ],
        out_specs=[
            pl.BlockSpec((N_MOL, 1), lambda i: (0, 0)),
            pl.BlockSpec((BE, 4), lambda i: (i, 0)),
        ],
        out_shape=[
            jax.ShapeDtypeStruct((N_MOL, 1), jnp.float32),
            jax.ShapeDtypeStruct((N_EDGES, 4), jnp.float32),
        ],
    )(h_pad, m_forces, V_st, W_energy, W_forces, b2, bid_pad)

    idx3 = idx_t.astype(jnp.int32).reshape(NW, NCH, CH)
    zeros = jnp.zeros((NPAD, 4), jnp.float32)

    partials = pl.kernel(
        _sc_body,
        out_type=jax.ShapeDtypeStruct((NC, NPAD, 4), jnp.float32),
        mesh=plsc.VectorSubcoreMesh(core_axis_name="c", subcore_axis_name="s"),
        scratch_types=[
            pltpu.VMEM((EPW, 4), jnp.float32),
            pltpu.VMEM((NCH, CH), jnp.int32),
            pltpu.VMEM_SHARED((NPAD, 4), jnp.float32),
        ],
    )(fv, idx3, zeros)

    forces = pl.pallas_call(
        _comb_body,
        out_shape=jax.ShapeDtypeStruct((N_NODES, 3), jnp.float32),
    )(partials)

    return (energy2.reshape(-1), forces)


# speedup vs baseline: 2.0494x; 2.0494x over previous
import jax
import jax.numpy as jnp
from jax import lax
from jax.experimental import pallas as pl
from jax.experimental.pallas import tpu as pltpu
from jax.experimental.pallas import tpu_sc as plsc

N_NODES = 10000
N_EDGES = 320000
D = 128
N_MOL = 64

BE = 2000
GRID = N_EDGES // BE
HN = 64
N_HPAD = GRID * HN

NC = 2
NS = 16
NW = NC * NS
EPW = N_EDGES // NW
NV = EPW // 16
AR = 960
AC = 32
RCH = 96
NRCH = AR // RCH
ORPW = AR // NS


def _tc_body(h_ref, m_ref, v_ref, we_ref, wf_ref, b_ref, bid_ref,
             e_ref, fv_ref):
    f = jnp.dot(m_ref[...], wf_ref[...],
                preferred_element_type=jnp.float32) + b_ref[0, 0]
    fv_ref[...] = f * v_ref[...]

    e = jnp.dot(h_ref[...], we_ref[...],
                preferred_element_type=jnp.float32)
    ids = bid_ref[0, 0, :]
    seg = lax.broadcasted_iota(jnp.int32, (N_MOL, HN), 0)
    onehot = (seg == ids[None, :]).astype(jnp.float32)
    contrib = jnp.dot(onehot, e, preferred_element_type=jnp.float32)

    @pl.when(pl.program_id(0) == 0)
    def _():
        e_ref[...] = jnp.zeros_like(e_ref)

    e_ref[...] += contrib


def _sc_body(fv_hbm, idx_hbm, zeros_hbm, ar_hbm, out_hbm,
             pk_v, idx_v, acc_v, ar_v, acc_sh):
    c = lax.axis_index("c")
    s = lax.axis_index("s")
    w = c * NS + s

    pltpu.sync_copy(fv_hbm.at[pl.ds(w * EPW, EPW)], pk_v)
    pltpu.sync_copy(idx_hbm.at[pl.ds(w * EPW, EPW)], idx_v)
    pltpu.sync_copy(ar_hbm, ar_v)
    pltpu.sync_copy(zeros_hbm, acc_v)

    @pl.when(s == 0)
    def _():
        pltpu.sync_copy(zeros_hbm, acc_sh)

    lanes = lax.iota(jnp.int32, 16)
    czero = jnp.zeros((16,), jnp.int32)
    cone = jnp.full((16,), 1, jnp.int32)
    ctwo = jnp.full((16,), 2, jnp.int32)

    def step(i, carry):
        idxv = idx_v[pl.ds(i * 16, 16)]
        rows = lanes + i * 16
        dst0 = idxv * 3
        dst1 = dst0 + 1
        dst2 = dst0 + 2
        v0 = plsc.load_gather(pk_v, [rows, czero])
        v1 = plsc.load_gather(pk_v, [rows, cone])
        v2 = plsc.load_gather(pk_v, [rows, ctwo])
        plsc.addupdate_scatter(acc_v, [dst0 >> 5, dst0 & 31], v0)
        plsc.addupdate_scatter(acc_v, [dst1 >> 5, dst1 & 31], v1)
        plsc.addupdate_scatter(acc_v, [dst2 >> 5, dst2 & 31], v2)
        return carry

    lax.fori_loop(0, NV, step, 0)

    plsc.subcore_barrier()

    def red(g, carry):
        pltpu.sync_copy(acc_v.at[pl.ds(g * RCH, RCH)],
                        acc_sh.at[ar_v.at[g]], add=True)
        return carry

    lax.fori_loop(0, NRCH, red, 0)

    plsc.subcore_barrier()

    pltpu.sync_copy(acc_sh.at[pl.ds(s * ORPW, ORPW)],
                    out_hbm.at[c, pl.ds(s * ORPW, ORPW)])


def _comb_body(p_ref, o_ref):
    o_ref[...] = p_ref[0, :N_NODES, :] + p_ref[1, :N_NODES, :]


def kernel(h_energy, m_forces, V_st, W_energy, W_forces, b_forces,
           batch_ids, idx_t):
    h_pad = jnp.pad(h_energy, ((0, N_HPAD - N_NODES), (0, 0)))
    bid_pad = jnp.pad(batch_ids, (0, N_HPAD - N_NODES)).astype(jnp.int32)
    bid_pad = bid_pad.reshape(GRID, 1, HN)
    b2 = b_forces.reshape(1, 1)

    energy2, fv = pl.pallas_call(
        _tc_body,
        grid=(GRID,),
        in_specs=[
            pl.BlockSpec((HN, D), lambda i: (i, 0)),
            pl.BlockSpec((BE, D), lambda i: (i, 0)),
            pl.BlockSpec((BE, 3), lambda i: (i, 0)),
            pl.BlockSpec((D, 1), lambda i: (0, 0)),
            pl.BlockSpec((D, 1), lambda i: (0, 0)),
            pl.BlockSpec((1, 1), lambda i: (0, 0)),
            pl.BlockSpec((1, 1, HN), lambda i: (i, 0, 0)),
        ],
        out_specs=[
            pl.BlockSpec((N_MOL, 1), lambda i: (0, 0)),
            pl.BlockSpec((BE, 3), lambda i: (i, 0)),
        ],
        out_shape=[
            jax.ShapeDtypeStruct((N_MOL, 1), jnp.float32),
            jax.ShapeDtypeStruct((N_EDGES, 3), jnp.float32),
        ],
    )(h_pad, m_forces, V_st, W_energy, W_forces, b2, bid_pad)

    idx32 = idx_t.astype(jnp.int32)
    zeros = jnp.zeros((AR, AC), jnp.float32)
    ar = (jnp.arange(AR, dtype=jnp.int32)).reshape(NRCH, RCH)

    partials = pl.kernel(
        _sc_body,
        out_type=jax.ShapeDtypeStruct((NC, AR, AC), jnp.float32),
        mesh=plsc.VectorSubcoreMesh(core_axis_name="c", subcore_axis_name="s"),
        scratch_types=[
            pltpu.VMEM((EPW, 3), jnp.float32),
            pltpu.VMEM((EPW,), jnp.int32),
            pltpu.VMEM((AR, AC), jnp.float32),
            pltpu.VMEM((NRCH, RCH), jnp.int32),
            pltpu.VMEM_SHARED((AR, AC), jnp.float32),
        ],
        compiler_params=pltpu.CompilerParams(use_tc_tiling_on_sc=False,
                                             needs_layout_passes=False),
    )(fv, idx32, zeros, ar)

    p3 = partials.reshape(NC, AR * AC // 3, 3)

    forces = pl.pallas_call(
        _comb_body,
        out_shape=jax.ShapeDtypeStruct((N_NODES, 3), jnp.float32),
    )(p3)

    return (energy2.reshape(-1), forces)


# speedup vs baseline: 2.1275x; 1.0381x over previous
import jax
import jax.numpy as jnp
from jax import lax
from jax.experimental import pallas as pl
from jax.experimental.pallas import tpu as pltpu
from jax.experimental.pallas import tpu_sc as plsc

N_NODES = 10000
N_EDGES = 320000
D = 128
N_MOL = 64

BE = 512
GRID = N_EDGES // BE
HN = 16

NC = 2
NS = 16
NW = NC * NS
EPW = N_EDGES // NW
NV = EPW // 16
AR = 960
AC = 32
RCH = 96
NRCH = AR // RCH
ORPW = AR // NS


def _tc_body(h_ref, m_ref, we_ref, wf_ref, b_ref, bid_ref,
             e_ref, f_ref):
    f = jnp.dot(m_ref[...], wf_ref[...],
                preferred_element_type=jnp.float32) + b_ref[0, 0]
    f_ref[...] = f.reshape(BE)

    e = jnp.dot(h_ref[...], we_ref[...],
                preferred_element_type=jnp.float32)
    ids = bid_ref[0, 0, :]
    seg = lax.broadcasted_iota(jnp.int32, (N_MOL, HN), 0)
    onehot = (seg == ids[None, :]).astype(jnp.float32)
    contrib = jnp.dot(onehot, e, preferred_element_type=jnp.float32)

    @pl.when(pl.program_id(0) == 0)
    def _():
        e_ref[...] = jnp.zeros_like(e_ref)

    e_ref[...] += contrib


def _sc_body(f_hbm, vx_hbm, vy_hbm, vz_hbm, idx_hbm, zeros_hbm, ar_hbm,
             out_hbm, f_v, vx_v, vy_v, vz_v, idx_v, acc_v, ar_v, acc_sh):
    c = lax.axis_index("c")
    s = lax.axis_index("s")
    w = c * NS + s
    sl = pl.ds(w * EPW, EPW)

    pltpu.sync_copy(f_hbm.at[sl], f_v)
    pltpu.sync_copy(vx_hbm.at[sl], vx_v)
    pltpu.sync_copy(vy_hbm.at[sl], vy_v)
    pltpu.sync_copy(vz_hbm.at[sl], vz_v)
    pltpu.sync_copy(idx_hbm.at[sl], idx_v)
    pltpu.sync_copy(zeros_hbm, acc_v)
    pltpu.sync_copy(ar_hbm, ar_v)

    @pl.when(s == 0)
    def _():
        pltpu.sync_copy(zeros_hbm, acc_sh)

    def step(i, carry):
        lane = pl.ds(i * 16, 16)
        idxv = idx_v[lane]
        fv = f_v[lane]
        dst0 = idxv * 3
        dst1 = dst0 + 1
        dst2 = dst0 + 2
        plsc.addupdate_scatter(acc_v, [dst0 >> 5, dst0 & 31], fv * vx_v[lane])
        plsc.addupdate_scatter(acc_v, [dst1 >> 5, dst1 & 31], fv * vy_v[lane])
        plsc.addupdate_scatter(acc_v, [dst2 >> 5, dst2 & 31], fv * vz_v[lane])
        return carry

    lax.fori_loop(0, NV, step, 0)

    plsc.subcore_barrier()

    def red(g, carry):
        pltpu.sync_copy(acc_v.at[pl.ds(g * RCH, RCH)],
                        acc_sh.at[ar_v.at[g]], add=True)
        return carry

    lax.fori_loop(0, NRCH, red, 0)

    plsc.subcore_barrier()

    pltpu.sync_copy(acc_sh.at[pl.ds(s * ORPW, ORPW)],
                    out_hbm.at[c, pl.ds(s * ORPW, ORPW)])


def _comb_body(p_ref, o_ref):
    o_ref[...] = p_ref[0, :N_NODES, :] + p_ref[1, :N_NODES, :]


def kernel(h_energy, m_forces, V_st, W_energy, W_forces, b_forces,
           batch_ids, idx_t):
    bid3 = batch_ids.astype(jnp.int32).reshape(GRID, 1, HN)
    b2 = b_forces.reshape(1, 1)

    energy2, f = pl.pallas_call(
        _tc_body,
        grid=(GRID,),
        in_specs=[
            pl.BlockSpec((HN, D), lambda i: (i, 0)),
            pl.BlockSpec((BE, D), lambda i: (i, 0)),
            pl.BlockSpec((D, 1), lambda i: (0, 0)),
            pl.BlockSpec((D, 1), lambda i: (0, 0)),
            pl.BlockSpec((1, 1), lambda i: (0, 0)),
            pl.BlockSpec((1, 1, HN), lambda i: (i, 0, 0)),
        ],
        out_specs=[
            pl.BlockSpec((N_MOL, 1), lambda i: (0, 0)),
            pl.BlockSpec((BE,), lambda i: (i,)),
        ],
        out_shape=[
            jax.ShapeDtypeStruct((N_MOL, 1), jnp.float32),
            jax.ShapeDtypeStruct((N_EDGES,), jnp.float32),
        ],
    )(h_energy, m_forces, W_energy, W_forces, b2, bid3)

    vx = V_st[:, 0]
    vy = V_st[:, 1]
    vz = V_st[:, 2]
    idx32 = idx_t.astype(jnp.int32)
    zeros = jnp.zeros((AR, AC), jnp.float32)
    ar = (jnp.arange(AR, dtype=jnp.int32)).reshape(NRCH, RCH)

    partials = pl.kernel(
        _sc_body,
        out_type=jax.ShapeDtypeStruct((NC, AR, AC), jnp.float32),
        mesh=plsc.VectorSubcoreMesh(core_axis_name="c", subcore_axis_name="s"),
        scratch_types=[
            pltpu.VMEM((EPW,), jnp.float32),
            pltpu.VMEM((EPW,), jnp.float32),
            pltpu.VMEM((EPW,), jnp.float32),
            pltpu.VMEM((EPW,), jnp.float32),
            pltpu.VMEM((EPW,), jnp.int32),
            pltpu.VMEM((AR, AC), jnp.float32),
            pltpu.VMEM((NRCH, RCH), jnp.int32),
            pltpu.VMEM_SHARED((AR, AC), jnp.float32),
        ],
        compiler_params=pltpu.CompilerParams(use_tc_tiling_on_sc=False,
                                             needs_layout_passes=False),
    )(f, vx, vy, vz, idx32, zeros, ar)

    p3 = partials.reshape(NC, AR * AC // 3, 3)

    forces = pl.pallas_call(
        _comb_body,
        out_shape=jax.ShapeDtypeStruct((N_NODES, 3), jnp.float32),
    )(p3)

    return (energy2.reshape(-1), forces)


# speedup vs baseline: 2.1299x; 1.0011x over previous
import jax
import jax.numpy as jnp
from jax import lax
from jax.experimental import pallas as pl
from jax.experimental.pallas import tpu as pltpu
from jax.experimental.pallas import tpu_sc as plsc

N_NODES = 10000
N_EDGES = 320000
D = 128
N_MOL = 64

BE = 512
GRID = N_EDGES // BE
HN = 16

NC = 2
NS = 16
NW = NC * NS
EPW = N_EDGES // NW
NV = EPW // 16
AR = 960
AC = 32
RCH = 96
NRCH = AR // RCH
ORPW = AR // NS


def _tc_body(h_ref, m_ref, we_ref, wf_ref, b_ref, bid_ref,
             e_ref, f_ref):
    f = jnp.dot(m_ref[...], wf_ref[...],
                preferred_element_type=jnp.float32) + b_ref[0, 0]
    f_ref[...] = f.reshape(BE)

    e = jnp.dot(h_ref[...], we_ref[...],
                preferred_element_type=jnp.float32)
    ids = bid_ref[0, 0, :]
    seg = lax.broadcasted_iota(jnp.int32, (N_MOL, HN), 0)
    onehot = (seg == ids[None, :]).astype(jnp.float32)
    contrib = jnp.dot(onehot, e, preferred_element_type=jnp.float32)

    @pl.when(pl.program_id(0) == 0)
    def _():
        e_ref[...] = jnp.zeros_like(e_ref)

    e_ref[...] += contrib


def _sc_body(f_hbm, vx_hbm, vy_hbm, vz_hbm, idx_hbm, zeros_hbm, ar_hbm,
             out_hbm, f_v, vx_v, vy_v, vz_v, idx_v, acc_v, ar_v, acc_sh):
    c = lax.axis_index("c")
    s = lax.axis_index("s")
    w = c * NS + s
    sl = pl.ds(w * EPW, EPW)

    pltpu.sync_copy(f_hbm.at[sl], f_v)
    pltpu.sync_copy(vx_hbm.at[sl], vx_v)
    pltpu.sync_copy(vy_hbm.at[sl], vy_v)
    pltpu.sync_copy(vz_hbm.at[sl], vz_v)
    pltpu.sync_copy(idx_hbm.at[sl], idx_v)
    pltpu.sync_copy(ar_hbm, ar_v)

    @pl.when(s == 0)
    def _():
        pltpu.sync_copy(zeros_hbm, acc_sh)

    zvec = jnp.zeros((16,), jnp.float32)

    def zero(j, carry):
        acc_v[j, pl.ds(0, 16)] = zvec
        acc_v[j, pl.ds(16, 16)] = zvec
        return carry

    lax.fori_loop(0, AR, zero, 0)

    def step(i, carry):
        lane = pl.ds(i * 16, 16)
        idxv = idx_v[lane]
        fv = f_v[lane]
        dst0 = idxv * 3
        dst1 = dst0 + 1
        dst2 = dst0 + 2
        plsc.addupdate_scatter(acc_v, [dst0 >> 5, dst0 & 31], fv * vx_v[lane])
        plsc.addupdate_scatter(acc_v, [dst1 >> 5, dst1 & 31], fv * vy_v[lane])
        plsc.addupdate_scatter(acc_v, [dst2 >> 5, dst2 & 31], fv * vz_v[lane])
        return carry

    lax.fori_loop(0, NV, step, 0)

    plsc.subcore_barrier()

    def red(g, carry):
        pltpu.sync_copy(acc_v.at[pl.ds(g * RCH, RCH)],
                        acc_sh.at[ar_v.at[g]], add=True)
        return carry

    lax.fori_loop(0, NRCH, red, 0)

    plsc.subcore_barrier()

    pltpu.sync_copy(acc_sh.at[pl.ds(s * ORPW, ORPW)],
                    out_hbm.at[c, pl.ds(s * ORPW, ORPW)])


def _comb_body(p_ref, o_ref):
    o_ref[...] = p_ref[0, :N_NODES, :] + p_ref[1, :N_NODES, :]


def kernel(h_energy, m_forces, V_st, W_energy, W_forces, b_forces,
           batch_ids, idx_t):
    bid3 = batch_ids.astype(jnp.int32).reshape(GRID, 1, HN)
    b2 = b_forces.reshape(1, 1)

    energy2, f = pl.pallas_call(
        _tc_body,
        grid=(GRID,),
        in_specs=[
            pl.BlockSpec((HN, D), lambda i: (i, 0)),
            pl.BlockSpec((BE, D), lambda i: (i, 0)),
            pl.BlockSpec((D, 1), lambda i: (0, 0)),
            pl.BlockSpec((D, 1), lambda i: (0, 0)),
            pl.BlockSpec((1, 1), lambda i: (0, 0)),
            pl.BlockSpec((1, 1, HN), lambda i: (i, 0, 0)),
        ],
        out_specs=[
            pl.BlockSpec((N_MOL, 1), lambda i: (0, 0)),
            pl.BlockSpec((BE,), lambda i: (i,)),
        ],
        out_shape=[
            jax.ShapeDtypeStruct((N_MOL, 1), jnp.float32),
            jax.ShapeDtypeStruct((N_EDGES,), jnp.float32),
        ],
    )(h_energy, m_forces, W_energy, W_forces, b2, bid3)

    vx = V_st[:, 0]
    vy = V_st[:, 1]
    vz = V_st[:, 2]
    idx32 = idx_t.astype(jnp.int32)
    zeros = jnp.zeros((AR, AC), jnp.float32)
    ar = (jnp.arange(AR, dtype=jnp.int32)).reshape(NRCH, RCH)

    partials = pl.kernel(
        _sc_body,
        out_type=jax.ShapeDtypeStruct((NC, AR, AC), jnp.float32),
        mesh=plsc.VectorSubcoreMesh(core_axis_name="c", subcore_axis_name="s"),
        scratch_types=[
            pltpu.VMEM((EPW,), jnp.float32),
            pltpu.VMEM((EPW,), jnp.float32),
            pltpu.VMEM((EPW,), jnp.float32),
            pltpu.VMEM((EPW,), jnp.float32),
            pltpu.VMEM((EPW,), jnp.int32),
            pltpu.VMEM((AR, AC), jnp.float32),
            pltpu.VMEM((NRCH, RCH), jnp.int32),
            pltpu.VMEM_SHARED((AR, AC), jnp.float32),
        ],
        compiler_params=pltpu.CompilerParams(use_tc_tiling_on_sc=False,
                                             needs_layout_passes=False),
    )(f, vx, vy, vz, idx32, zeros, ar)

    p3 = partials.reshape(NC, AR * AC // 3, 3)

    forces = pl.pallas_call(
        _comb_body,
        out_shape=jax.ShapeDtypeStruct((N_NODES, 3), jnp.float32),
    )(p3)

    return (energy2.reshape(-1), forces)


# speedup vs baseline: 2.1499x; 1.0094x over previous
import functools

import jax
import jax.numpy as jnp
from jax import lax
from jax.experimental import pallas as pl
from jax.experimental.pallas import tpu as pltpu
from jax.experimental.pallas import tpu_sc as plsc

N_NODES = 10000
N_EDGES = 320000
D = 128
N_MOL = 64

BE = 512
GRID1 = 313
GRID2 = 312
E1 = GRID1 * BE
E2 = GRID2 * BE
HN = 1000
EGRID = N_NODES // HN

NC = 2
NS = 16
NW = NC * NS
EPW1 = E1 // NW
EPW2 = E2 // NW
AR = 960
AC = 32
RCH = 96
NRCH = AR // RCH
ORPW = AR // NS


def _te_body(h_ref, we_ref, bid_ref, e_ref):
    e = jnp.dot(h_ref[...], we_ref[...],
                preferred_element_type=jnp.float32)
    ids = bid_ref[0, 0, :]
    seg = lax.broadcasted_iota(jnp.int32, (N_MOL, HN), 0)
    onehot = (seg == ids[None, :]).astype(jnp.float32)
    contrib = jnp.dot(onehot, e, preferred_element_type=jnp.float32)

    @pl.when(pl.program_id(0) == 0)
    def _():
        e_ref[...] = jnp.zeros_like(e_ref)

    e_ref[...] += contrib


def _tf_body(m_ref, wf_ref, b_ref, f_ref):
    f = jnp.dot(m_ref[...], wf_ref[...],
                preferred_element_type=jnp.float32) + b_ref[0, 0]
    f_ref[...] = f.reshape(BE)


def _make_sc_body(epw, off):
    nv = epw // 16

    def _sc_body(f_hbm, vx_hbm, vy_hbm, vz_hbm, idx_hbm, zeros_hbm, ar_hbm,
                 out_hbm, f_v, vx_v, vy_v, vz_v, idx_v, acc_v, ar_v, acc_sh):
        c = lax.axis_index("c")
        s = lax.axis_index("s")
        w = c * NS + s
        sl = pl.ds(off + w * epw, epw)

        pltpu.sync_copy(f_hbm.at[pl.ds(w * epw, epw)], f_v)
        pltpu.sync_copy(vx_hbm.at[sl], vx_v)
        pltpu.sync_copy(vy_hbm.at[sl], vy_v)
        pltpu.sync_copy(vz_hbm.at[sl], vz_v)
        pltpu.sync_copy(idx_hbm.at[sl], idx_v)
        pltpu.sync_copy(ar_hbm, ar_v)

        @pl.when(s == 0)
        def _():
            pltpu.sync_copy(zeros_hbm, acc_sh)

        zvec = jnp.zeros((16,), jnp.float32)

        def zero(j, carry):
            acc_v[j, pl.ds(0, 16)] = zvec
            acc_v[j, pl.ds(16, 16)] = zvec
            return carry

        lax.fori_loop(0, AR, zero, 0)

        def step(i, carry):
            lane = pl.ds(i * 16, 16)
            idxv = idx_v[lane]
            fv = f_v[lane]
            dst0 = idxv * 3
            dst1 = dst0 + 1
            dst2 = dst0 + 2
            plsc.addupdate_scatter(acc_v, [dst0 >> 5, dst0 & 31],
                                   fv * vx_v[lane])
            plsc.addupdate_scatter(acc_v, [dst1 >> 5, dst1 & 31],
                                   fv * vy_v[lane])
            plsc.addupdate_scatter(acc_v, [dst2 >> 5, dst2 & 31],
                                   fv * vz_v[lane])
            return carry

        lax.fori_loop(0, nv, step, 0)

        plsc.subcore_barrier()

        def red(g, carry):
            pltpu.sync_copy(acc_v.at[pl.ds(g * RCH, RCH)],
                            acc_sh.at[ar_v.at[g]], add=True)
            return carry

        lax.fori_loop(0, NRCH, red, 0)

        plsc.subcore_barrier()

        pltpu.sync_copy(acc_sh.at[pl.ds(s * ORPW, ORPW)],
                        out_hbm.at[c, pl.ds(s * ORPW, ORPW)])

    return _sc_body


def _comb_body(p_ref, q_ref, o_ref):
    o_ref[...] = (p_ref[0, :N_NODES, :] + p_ref[1, :N_NODES, :] +
                  q_ref[0, :N_NODES, :] + q_ref[1, :N_NODES, :])


def _sc_scatter(f_chunk, vx, vy, vz, idx32, zeros, ar, epw, off):
    return pl.kernel(
        _make_sc_body(epw, off),
        out_type=jax.ShapeDtypeStruct((NC, AR, AC), jnp.float32),
        mesh=plsc.VectorSubcoreMesh(core_axis_name="c", subcore_axis_name="s"),
        scratch_types=[
            pltpu.VMEM((epw,), jnp.float32),
            pltpu.VMEM((epw,), jnp.float32),
            pltpu.VMEM((epw,), jnp.float32),
            pltpu.VMEM((epw,), jnp.float32),
            pltpu.VMEM((epw,), jnp.int32),
            pltpu.VMEM((AR, AC), jnp.float32),
            pltpu.VMEM((NRCH, RCH), jnp.int32),
            pltpu.VMEM_SHARED((AR, AC), jnp.float32),
        ],
        compiler_params=pltpu.CompilerParams(use_tc_tiling_on_sc=False,
                                             needs_layout_passes=False),
    )(f_chunk, vx, vy, vz, idx32, zeros, ar)


def kernel(h_energy, m_forces, V_st, W_energy, W_forces, b_forces,
           batch_ids, idx_t):
    bid3 = batch_ids.astype(jnp.int32).reshape(EGRID, 1, HN)
    b2 = b_forces.reshape(1, 1)

    energy2 = pl.pallas_call(
        _te_body,
        grid=(EGRID,),
        in_specs=[
            pl.BlockSpec((HN, D), lambda i: (i, 0)),
            pl.BlockSpec((D, 1), lambda i: (0, 0)),
            pl.BlockSpec((1, 1, HN), lambda i: (i, 0, 0)),
        ],
        out_specs=pl.BlockSpec((N_MOL, 1), lambda i: (0, 0)),
        out_shape=jax.ShapeDtypeStruct((N_MOL, 1), jnp.float32),
    )(h_energy, W_energy, bid3)

    def tf_call(grid, base):
        return pl.pallas_call(
            _tf_body,
            grid=(grid,),
            in_specs=[
                pl.BlockSpec((BE, D), lambda i: (i + base, 0)),
                pl.BlockSpec((D, 1), lambda i: (0, 0)),
                pl.BlockSpec((1, 1), lambda i: (0, 0)),
            ],
            out_specs=pl.BlockSpec((BE,), lambda i: (i,)),
            out_shape=jax.ShapeDtypeStruct((grid * BE,), jnp.float32),
        )(m_forces, W_forces, b2)

    f1 = tf_call(GRID1, 0)
    f2 = tf_call(GRID2, GRID1)

    vx = V_st[:, 0]
    vy = V_st[:, 1]
    vz = V_st[:, 2]
    idx32 = idx_t.astype(jnp.int32)
    zeros = jnp.zeros((AR, AC), jnp.float32)
    ar = (jnp.arange(AR, dtype=jnp.int32)).reshape(NRCH, RCH)

    p1 = _sc_scatter(f1, vx, vy, vz, idx32, zeros, ar, EPW1, 0)
    p2 = _sc_scatter(f2, vx, vy, vz, idx32, zeros, ar, EPW2, E1)

    q1 = p1.reshape(NC, AR * AC // 3, 3)
    q2 = p2.reshape(NC, AR * AC // 3, 3)

    forces = pl.pallas_call(
        _comb_body,
        out_shape=jax.ShapeDtypeStruct((N_NODES, 3), jnp.float32),
    )(q1, q2)

    return (energy2.reshape(-1), forces)


# speedup vs baseline: 5.2784x; 2.4552x over previous
import functools

import jax
import jax.numpy as jnp
from jax import lax
from jax.experimental import pallas as pl
from jax.experimental.pallas import tpu as pltpu
from jax.experimental.pallas import tpu_sc as plsc

N_NODES = 10000
N_EDGES = 320000
D = 128
N_MOL = 64

BE = 2560
GRID1 = 64
GRID2 = 61
E1 = GRID1 * BE
E2 = GRID2 * BE
HN = 1000
EGRID = N_NODES // HN

NC = 2
NS = 16
NW = NC * NS
EPW1 = E1 // NW
EPW2 = E2 // NW
AR = 960
AC = 32
RCH = 96
NRCH = AR // RCH
ORPW = AR // NS


def _te_body(h_ref, we_ref, bid_ref, e_ref):
    e = jnp.dot(h_ref[...], we_ref[...],
                preferred_element_type=jnp.float32)
    ids = bid_ref[0, 0, :]
    seg = lax.broadcasted_iota(jnp.int32, (N_MOL, HN), 0)
    onehot = (seg == ids[None, :]).astype(jnp.float32)
    contrib = jnp.dot(onehot, e, preferred_element_type=jnp.float32)

    @pl.when(pl.program_id(0) == 0)
    def _():
        e_ref[...] = jnp.zeros_like(e_ref)

    e_ref[...] += contrib


def _tf_body(m_ref, wf_ref, b_ref, f_ref):
    f = jnp.dot(m_ref[...], wf_ref[...],
                preferred_element_type=jnp.float32) + b_ref[0, 0]
    f_ref[...] = f.reshape(1, 4, BE // 4)


def _make_sc_body(epw, off):
    nv = epw // 16

    def _sc_body(f_hbm, vx_hbm, vy_hbm, vz_hbm, idx_hbm, zeros_hbm, ar_hbm,
                 out_hbm, f_v, vx_v, vy_v, vz_v, idx_v, acc_v, ar_v, acc_sh):
        c = lax.axis_index("c")
        s = lax.axis_index("s")
        w = c * NS + s
        sl = pl.ds(off + w * epw, epw)

        pltpu.sync_copy(f_hbm.at[pl.ds(w * epw, epw)], f_v)
        pltpu.sync_copy(vx_hbm.at[sl], vx_v)
        pltpu.sync_copy(vy_hbm.at[sl], vy_v)
        pltpu.sync_copy(vz_hbm.at[sl], vz_v)
        pltpu.sync_copy(idx_hbm.at[sl], idx_v)
        pltpu.sync_copy(ar_hbm, ar_v)

        @pl.when(s == 0)
        def _():
            pltpu.sync_copy(zeros_hbm, acc_sh)

        zvec = jnp.zeros((16,), jnp.float32)

        def zero(j, carry):
            acc_v[j, pl.ds(0, 16)] = zvec
            acc_v[j, pl.ds(16, 16)] = zvec
            return carry

        lax.fori_loop(0, AR, zero, 0)

        def step(i, carry):
            lane = pl.ds(i * 16, 16)
            idxv = idx_v[lane]
            fv = f_v[lane]
            dst0 = idxv * 3
            dst1 = dst0 + 1
            dst2 = dst0 + 2
            plsc.addupdate_scatter(acc_v, [dst0 >> 5, dst0 & 31],
                                   fv * vx_v[lane])
            plsc.addupdate_scatter(acc_v, [dst1 >> 5, dst1 & 31],
                                   fv * vy_v[lane])
            plsc.addupdate_scatter(acc_v, [dst2 >> 5, dst2 & 31],
                                   fv * vz_v[lane])
            return carry

        lax.fori_loop(0, nv, step, 0)

        plsc.subcore_barrier()

        def red(g, carry):
            pltpu.sync_copy(acc_v.at[pl.ds(g * RCH, RCH)],
                            acc_sh.at[ar_v.at[g]], add=True)
            return carry

        lax.fori_loop(0, NRCH, red, 0)

        plsc.subcore_barrier()

        pltpu.sync_copy(acc_sh.at[pl.ds(s * ORPW, ORPW)],
                        out_hbm.at[c, pl.ds(s * ORPW, ORPW)])

    return _sc_body


def _comb_body(p_ref, q_ref, o_ref):
    o_ref[...] = (p_ref[0, :N_NODES, :] + p_ref[1, :N_NODES, :] +
                  q_ref[0, :N_NODES, :] + q_ref[1, :N_NODES, :])


def _sc_scatter(f_chunk, vx, vy, vz, idx32, zeros, ar, epw, off):
    return pl.kernel(
        _make_sc_body(epw, off),
        out_type=jax.ShapeDtypeStruct((NC, AR, AC), jnp.float32),
        mesh=plsc.VectorSubcoreMesh(core_axis_name="c", subcore_axis_name="s"),
        scratch_types=[
            pltpu.VMEM((epw,), jnp.float32),
            pltpu.VMEM((epw,), jnp.float32),
            pltpu.VMEM((epw,), jnp.float32),
            pltpu.VMEM((epw,), jnp.float32),
            pltpu.VMEM((epw,), jnp.int32),
            pltpu.VMEM((AR, AC), jnp.float32),
            pltpu.VMEM((NRCH, RCH), jnp.int32),
            pltpu.VMEM_SHARED((AR, AC), jnp.float32),
        ],
        compiler_params=pltpu.CompilerParams(use_tc_tiling_on_sc=False,
                                             needs_layout_passes=False),
    )(f_chunk, vx, vy, vz, idx32, zeros, ar)


def kernel(h_energy, m_forces, V_st, W_energy, W_forces, b_forces,
           batch_ids, idx_t):
    bid3 = batch_ids.astype(jnp.int32).reshape(EGRID, 1, HN)
    b2 = b_forces.reshape(1, 1)

    energy2 = pl.pallas_call(
        _te_body,
        grid=(EGRID,),
        in_specs=[
            pl.BlockSpec((HN, D), lambda i: (i, 0)),
            pl.BlockSpec((D, 1), lambda i: (0, 0)),
            pl.BlockSpec((1, 1, HN), lambda i: (i, 0, 0)),
        ],
        out_specs=pl.BlockSpec((N_MOL, 1), lambda i: (0, 0)),
        out_shape=jax.ShapeDtypeStruct((N_MOL, 1), jnp.float32),
    )(h_energy, W_energy, bid3)

    def tf_call(grid, base):
        return pl.pallas_call(
            _tf_body,
            grid=(grid,),
            in_specs=[
                pl.BlockSpec((BE, D), lambda i: (i + base, 0)),
                pl.BlockSpec((D, 1), lambda i: (0, 0)),
                pl.BlockSpec((1, 1), lambda i: (0, 0)),
            ],
            out_specs=pl.BlockSpec((1, 4, BE // 4), lambda i: (i, 0, 0)),
            out_shape=jax.ShapeDtypeStruct((grid, 4, BE // 4), jnp.float32),
        )(m_forces, W_forces, b2)

    f1 = tf_call(GRID1, 0).reshape(-1)
    f2 = tf_call(GRID2, GRID1).reshape(-1)

    vx = V_st[:, 0]
    vy = V_st[:, 1]
    vz = V_st[:, 2]
    idx32 = idx_t.astype(jnp.int32)
    zeros = jnp.zeros((AR, AC), jnp.float32)
    ar = (jnp.arange(AR, dtype=jnp.int32)).reshape(NRCH, RCH)

    p1 = _sc_scatter(f1, vx, vy, vz, idx32, zeros, ar, EPW1, 0)
    p2 = _sc_scatter(f2, vx, vy, vz, idx32, zeros, ar, EPW2, E1)

    q1 = p1.reshape(NC, AR * AC // 3, 3)
    q2 = p2.reshape(NC, AR * AC // 3, 3)

    forces = pl.pallas_call(
        _comb_body,
        out_shape=jax.ShapeDtypeStruct((N_NODES, 3), jnp.float32),
    )(q1, q2)

    return (energy2.reshape(-1), forces)


# speedup vs baseline: 7.1031x; 1.3457x over previous
import functools

import jax
import jax.numpy as jnp
from jax import lax
from jax.experimental import pallas as pl
from jax.experimental.pallas import tpu as pltpu
from jax.experimental.pallas import tpu_sc as plsc

N_NODES = 10000
N_EDGES = 320000
D = 128
N_MOL = 64

BE = 12800
GRID1 = 13
GRID2 = 12
E1 = GRID1 * BE
E2 = GRID2 * BE
HN = 1000
EGRID = N_NODES // HN

NC = 2
NS = 16
NW = NC * NS
EPW1 = E1 // NW
EPW2 = E2 // NW
AR = 960
AC = 32
RCH = 96
NRCH = AR // RCH
ORPW = AR // NS


def _te_body(h_ref, we_ref, bid_ref, e_ref):
    e = jnp.dot(h_ref[...], we_ref[...],
                preferred_element_type=jnp.float32)
    ids = bid_ref[0, 0, :]
    seg = lax.broadcasted_iota(jnp.int32, (N_MOL, HN), 0)
    onehot = (seg == ids[None, :]).astype(jnp.float32)
    contrib = jnp.dot(onehot, e, preferred_element_type=jnp.float32)

    @pl.when(pl.program_id(0) == 0)
    def _():
        e_ref[...] = jnp.zeros_like(e_ref)

    e_ref[...] += contrib


def _tf_body(m_ref, wf_ref, b_ref, f_ref):
    f = jnp.dot(m_ref[...], wf_ref[...],
                preferred_element_type=jnp.float32) + b_ref[0, 0]
    f_ref[...] = f.reshape(1, 4, BE // 4)


def _make_sc_body(epw, off):
    nv = epw // 16

    def _sc_body(f_hbm, vx_hbm, vy_hbm, vz_hbm, idx_hbm, zeros_hbm, ar_hbm,
                 out_hbm, f_v, vx_v, vy_v, vz_v, idx_v, acc_v, ar_v, acc_sh):
        c = lax.axis_index("c")
        s = lax.axis_index("s")
        w = c * NS + s
        sl = pl.ds(off + w * epw, epw)

        pltpu.sync_copy(f_hbm.at[pl.ds(w * epw, epw)], f_v)
        pltpu.sync_copy(vx_hbm.at[sl], vx_v)
        pltpu.sync_copy(vy_hbm.at[sl], vy_v)
        pltpu.sync_copy(vz_hbm.at[sl], vz_v)
        pltpu.sync_copy(idx_hbm.at[sl], idx_v)
        pltpu.sync_copy(ar_hbm, ar_v)

        @pl.when(s == 0)
        def _():
            pltpu.sync_copy(zeros_hbm, acc_sh)

        zvec = jnp.zeros((16,), jnp.float32)

        def zero(j, carry):
            acc_v[j, pl.ds(0, 16)] = zvec
            acc_v[j, pl.ds(16, 16)] = zvec
            return carry

        lax.fori_loop(0, AR, zero, 0)

        def step(i, carry):
            lane = pl.ds(i * 16, 16)
            idxv = idx_v[lane]
            fv = f_v[lane]
            dst0 = idxv * 3
            dst1 = dst0 + 1
            dst2 = dst0 + 2
            plsc.addupdate_scatter(acc_v, [dst0 >> 5, dst0 & 31],
                                   fv * vx_v[lane])
            plsc.addupdate_scatter(acc_v, [dst1 >> 5, dst1 & 31],
                                   fv * vy_v[lane])
            plsc.addupdate_scatter(acc_v, [dst2 >> 5, dst2 & 31],
                                   fv * vz_v[lane])
            return carry

        lax.fori_loop(0, nv, step, 0)

        plsc.subcore_barrier()

        def red(g, carry):
            pltpu.sync_copy(acc_v.at[pl.ds(g * RCH, RCH)],
                            acc_sh.at[ar_v.at[g]], add=True)
            return carry

        lax.fori_loop(0, NRCH, red, 0)

        plsc.subcore_barrier()

        pltpu.sync_copy(acc_sh.at[pl.ds(s * ORPW, ORPW)],
                        out_hbm.at[c, pl.ds(s * ORPW, ORPW)])

    return _sc_body


def _comb_body(p_ref, q_ref, o_ref):
    o_ref[...] = (p_ref[0, :N_NODES, :] + p_ref[1, :N_NODES, :] +
                  q_ref[0, :N_NODES, :] + q_ref[1, :N_NODES, :])


def _sc_scatter(f_chunk, vx, vy, vz, idx32, zeros, ar, epw, off):
    return pl.kernel(
        _make_sc_body(epw, off),
        out_type=jax.ShapeDtypeStruct((NC, AR, AC), jnp.float32),
        mesh=plsc.VectorSubcoreMesh(core_axis_name="c", subcore_axis_name="s"),
        scratch_types=[
            pltpu.VMEM((epw,), jnp.float32),
            pltpu.VMEM((epw,), jnp.float32),
            pltpu.VMEM((epw,), jnp.float32),
            pltpu.VMEM((epw,), jnp.float32),
            pltpu.VMEM((epw,), jnp.int32),
            pltpu.VMEM((AR, AC), jnp.float32),
            pltpu.VMEM((NRCH, RCH), jnp.int32),
            pltpu.VMEM_SHARED((AR, AC), jnp.float32),
        ],
        compiler_params=pltpu.CompilerParams(use_tc_tiling_on_sc=False,
                                             needs_layout_passes=False),
    )(f_chunk, vx, vy, vz, idx32, zeros, ar)


def kernel(h_energy, m_forces, V_st, W_energy, W_forces, b_forces,
           batch_ids, idx_t):
    bid3 = batch_ids.astype(jnp.int32).reshape(EGRID, 1, HN)
    b2 = b_forces.reshape(1, 1)

    energy2 = pl.pallas_call(
        _te_body,
        grid=(EGRID,),
        in_specs=[
            pl.BlockSpec((HN, D), lambda i: (i, 0)),
            pl.BlockSpec((D, 1), lambda i: (0, 0)),
            pl.BlockSpec((1, 1, HN), lambda i: (i, 0, 0)),
        ],
        out_specs=pl.BlockSpec((N_MOL, 1), lambda i: (0, 0)),
        out_shape=jax.ShapeDtypeStruct((N_MOL, 1), jnp.float32),
    )(h_energy, W_energy, bid3)

    def tf_call(grid, base):
        return pl.pallas_call(
            _tf_body,
            grid=(grid,),
            in_specs=[
                pl.BlockSpec((BE, D), lambda i: (i + base, 0)),
                pl.BlockSpec((D, 1), lambda i: (0, 0)),
                pl.BlockSpec((1, 1), lambda i: (0, 0)),
            ],
            out_specs=pl.BlockSpec((1, 4, BE // 4), lambda i: (i, 0, 0)),
            out_shape=jax.ShapeDtypeStruct((grid, 4, BE // 4), jnp.float32),
        )(m_forces, W_forces, b2)

    f1 = tf_call(GRID1, 0).reshape(-1)
    f2 = tf_call(GRID2, GRID1).reshape(-1)

    vx = V_st[:, 0]
    vy = V_st[:, 1]
    vz = V_st[:, 2]
    idx32 = idx_t.astype(jnp.int32)
    zeros = jnp.zeros((AR, AC), jnp.float32)
    ar = (jnp.arange(AR, dtype=jnp.int32)).reshape(NRCH, RCH)

    p1 = _sc_scatter(f1, vx, vy, vz, idx32, zeros, ar, EPW1, 0)
    p2 = _sc_scatter(f2, vx, vy, vz, idx32, zeros, ar, EPW2, E1)

    q1 = p1.reshape(NC, AR * AC // 3, 3)
    q2 = p2.reshape(NC, AR * AC // 3, 3)

    forces = pl.pallas_call(
        _comb_body,
        out_shape=jax.ShapeDtypeStruct((N_NODES, 3), jnp.float32),
    )(q1, q2)

    return (energy2.reshape(-1), forces)


# speedup vs baseline: 7.1836x; 1.0113x over previous
import functools

import jax
import jax.numpy as jnp
from jax import lax
from jax.experimental import pallas as pl
from jax.experimental.pallas import tpu as pltpu
from jax.experimental.pallas import tpu_sc as plsc

N_NODES = 10000
N_EDGES = 320000
D = 128
N_MOL = 64

BE = 32000
GRID1 = 6
GRID2 = 4
E1 = GRID1 * BE
E2 = GRID2 * BE
HN = 1000
EGRID = N_NODES // HN

NC = 2
NS = 16
NW = NC * NS
EPW1 = E1 // NW
EPW2 = E2 // NW
AR = 960
AC = 32
RCH = 96
NRCH = AR // RCH
ORPW = AR // NS


def _te_body(h_ref, we_ref, bid_ref, e_ref):
    e = jnp.dot(h_ref[...], we_ref[...],
                preferred_element_type=jnp.float32)
    ids = bid_ref[0, 0, :]
    seg = lax.broadcasted_iota(jnp.int32, (N_MOL, HN), 0)
    onehot = (seg == ids[None, :]).astype(jnp.float32)
    contrib = jnp.dot(onehot, e, preferred_element_type=jnp.float32)

    @pl.when(pl.program_id(0) == 0)
    def _():
        e_ref[...] = jnp.zeros_like(e_ref)

    e_ref[...] += contrib


def _tf_body(m_ref, wf_ref, b_ref, f_ref):
    f = jnp.dot(m_ref[...], wf_ref[...],
                preferred_element_type=jnp.float32) + b_ref[0, 0]
    f_ref[...] = f.reshape(1, 25, BE // 25)


def _make_sc_body(epw, off):
    nv = epw // 16

    def _sc_body(f_hbm, vx_hbm, vy_hbm, vz_hbm, idx_hbm, zeros_hbm, ar_hbm,
                 out_hbm, f_v, vx_v, vy_v, vz_v, idx_v, acc_v, ar_v, acc_sh):
        c = lax.axis_index("c")
        s = lax.axis_index("s")
        w = c * NS + s
        sl = pl.ds(off + w * epw, epw)

        pltpu.sync_copy(f_hbm.at[pl.ds(w * epw, epw)], f_v)
        pltpu.sync_copy(vx_hbm.at[sl], vx_v)
        pltpu.sync_copy(vy_hbm.at[sl], vy_v)
        pltpu.sync_copy(vz_hbm.at[sl], vz_v)
        pltpu.sync_copy(idx_hbm.at[sl], idx_v)
        pltpu.sync_copy(ar_hbm, ar_v)

        @pl.when(s == 0)
        def _():
            pltpu.sync_copy(zeros_hbm, acc_sh)

        zvec = jnp.zeros((16,), jnp.float32)

        def zero(j, carry):
            acc_v[j, pl.ds(0, 16)] = zvec
            acc_v[j, pl.ds(16, 16)] = zvec
            return carry

        lax.fori_loop(0, AR, zero, 0)

        def step(i, carry):
            lane = pl.ds(i * 16, 16)
            idxv = idx_v[lane]
            fv = f_v[lane]
            dst0 = idxv * 3
            dst1 = dst0 + 1
            dst2 = dst0 + 2
            plsc.addupdate_scatter(acc_v, [dst0 >> 5, dst0 & 31],
                                   fv * vx_v[lane])
            plsc.addupdate_scatter(acc_v, [dst1 >> 5, dst1 & 31],
                                   fv * vy_v[lane])
            plsc.addupdate_scatter(acc_v, [dst2 >> 5, dst2 & 31],
                                   fv * vz_v[lane])
            return carry

        lax.fori_loop(0, nv, step, 0)

        plsc.subcore_barrier()

        def red(g, carry):
            pltpu.sync_copy(acc_v.at[pl.ds(g * RCH, RCH)],
                            acc_sh.at[ar_v.at[g]], add=True)
            return carry

        lax.fori_loop(0, NRCH, red, 0)

        plsc.subcore_barrier()

        pltpu.sync_copy(acc_sh.at[pl.ds(s * ORPW, ORPW)],
                        out_hbm.at[c, pl.ds(s * ORPW, ORPW)])

    return _sc_body


def _comb_body(p_ref, q_ref, o_ref):
    o_ref[...] = (p_ref[0, :N_NODES, :] + p_ref[1, :N_NODES, :] +
                  q_ref[0, :N_NODES, :] + q_ref[1, :N_NODES, :])


def _sc_scatter(f_chunk, vx, vy, vz, idx32, zeros, ar, epw, off):
    return pl.kernel(
        _make_sc_body(epw, off),
        out_type=jax.ShapeDtypeStruct((NC, AR, AC), jnp.float32),
        mesh=plsc.VectorSubcoreMesh(core_axis_name="c", subcore_axis_name="s"),
        scratch_types=[
            pltpu.VMEM((epw,), jnp.float32),
            pltpu.VMEM((epw,), jnp.float32),
            pltpu.VMEM((epw,), jnp.float32),
            pltpu.VMEM((epw,), jnp.float32),
            pltpu.VMEM((epw,), jnp.int32),
            pltpu.VMEM((AR, AC), jnp.float32),
            pltpu.VMEM((NRCH, RCH), jnp.int32),
            pltpu.VMEM_SHARED((AR, AC), jnp.float32),
        ],
        compiler_params=pltpu.CompilerParams(use_tc_tiling_on_sc=False,
                                             needs_layout_passes=False),
    )(f_chunk, vx, vy, vz, idx32, zeros, ar)


def kernel(h_energy, m_forces, V_st, W_energy, W_forces, b_forces,
           batch_ids, idx_t):
    bid3 = batch_ids.astype(jnp.int32).reshape(EGRID, 1, HN)
    b2 = b_forces.reshape(1, 1)

    energy2 = pl.pallas_call(
        _te_body,
        grid=(EGRID,),
        in_specs=[
            pl.BlockSpec((HN, D), lambda i: (i, 0)),
            pl.BlockSpec((D, 1), lambda i: (0, 0)),
            pl.BlockSpec((1, 1, HN), lambda i: (i, 0, 0)),
        ],
        out_specs=pl.BlockSpec((N_MOL, 1), lambda i: (0, 0)),
        out_shape=jax.ShapeDtypeStruct((N_MOL, 1), jnp.float32),
    )(h_energy, W_energy, bid3)

    def tf_call(grid, base):
        return pl.pallas_call(
            _tf_body,
            grid=(grid,),
            in_specs=[
                pl.BlockSpec((BE, D), lambda i: (i + base, 0)),
                pl.BlockSpec((D, 1), lambda i: (0, 0)),
                pl.BlockSpec((1, 1), lambda i: (0, 0)),
            ],
            out_specs=pl.BlockSpec((1, 25, BE // 25), lambda i: (i, 0, 0)),
            out_shape=jax.ShapeDtypeStruct((grid, 25, BE // 25), jnp.float32),
        )(m_forces, W_forces, b2)

    f1 = tf_call(GRID1, 0).reshape(-1)
    f2 = tf_call(GRID2, GRID1).reshape(-1)

    vx = V_st[:, 0]
    vy = V_st[:, 1]
    vz = V_st[:, 2]
    idx32 = idx_t.astype(jnp.int32)
    zeros = jnp.zeros((AR, AC), jnp.float32)
    ar = (jnp.arange(AR, dtype=jnp.int32)).reshape(NRCH, RCH)

    p1 = _sc_scatter(f1, vx, vy, vz, idx32, zeros, ar, EPW1, 0)
    p2 = _sc_scatter(f2, vx, vy, vz, idx32, zeros, ar, EPW2, E1)

    q1 = p1.reshape(NC, AR * AC // 3, 3)
    q2 = p2.reshape(NC, AR * AC // 3, 3)

    forces = pl.pallas_call(
        _comb_body,
        out_shape=jax.ShapeDtypeStruct((N_NODES, 3), jnp.float32),
    )(q1, q2)

    return (energy2.reshape(-1), forces)
